# baseline (speedup 1.0000x reference)
import functools

import jax
import jax.numpy as jnp
from jax import lax
from jax.experimental import pallas as pl
from jax.experimental.pallas import tpu as pltpu
from jax.experimental.pallas import tpu_sc as plsc

N_WAY = 5
K_SHOT = 5
TEMP = 2.0
MS = 500
MSP = 512
M = 100
QB = 5
R = QB * M
RP = 512
NW = 32


def _stage1_body(nb, sup_ref, qry_ref, kcol_ref, dff_ref, cm_ref, buf_ref):
    t = pl.program_id(0)
    wslot = t % 2
    rslot = (t + 1) % 2

    sup = sup_ref[0]
    snorm = jnp.sqrt(jnp.sum(sup * sup, axis=0, keepdims=True))
    sn = sup / (snorm + 1e-8)
    qv = qry_ref[0]
    qnorm = jnp.sqrt(jnp.sum(qv * qv, axis=1, keepdims=True))
    qn = qv / (qnorm + 1e-8)
    colpad = lax.broadcasted_iota(jnp.int32, (R, MSP), 1) >= MS
    cif = lax.broadcasted_iota(jnp.int32, (R, MSP), 1).astype(jnp.float32)

    cm_cols = []
    am_cols = []
    for n in range(N_WAY):
        s_old = buf_ref[rslot, :, n * MSP:(n + 1) * MSP]
        rmax_n = jnp.max(s_old, axis=1, keepdims=True)
        am_n = jnp.min(jnp.where(s_old == rmax_n, cif, float(MSP)),
                       axis=1, keepdims=True)
        cm_cols.append(rmax_n)
        am_cols.append(am_n)
        s_new = lax.dot_general(qn, sn[:, n * MSP:(n + 1) * MSP],
                                (((1,), (0,)), ((), ())),
                                precision=lax.Precision.HIGHEST,
                                preferred_element_type=jnp.float32)
        buf_ref[wslot, :, n * MSP:(n + 1) * MSP] = \
            jnp.where(colpad, -2.0, s_new)

    cm = jnp.concatenate(cm_cols, axis=1)
    t1 = jnp.max(cm, axis=1, keepdims=True)
    i5 = lax.broadcasted_iota(jnp.int32, (R, N_WAY), 1).astype(jnp.float32)
    first = jnp.min(jnp.where(cm == t1, i5, float(N_WAY)),
                    axis=1, keepdims=True)
    t2 = jnp.max(jnp.where(i5 == first, -3.0, cm), axis=1, keepdims=True)
    dff = t1 - t2

    kcol = jnp.full((R, 1), jnp.float32(N_WAY * MSP))
    for n in range(N_WAY):
        cand = jnp.where(cm[:, n:n + 1] == t1, am_cols[n] + float(n * MSP),
                         jnp.float32(N_WAY * MSP))
        kcol = jnp.minimum(kcol, cand)

    kcol_ref[0, :R, :] = kcol
    dff_ref[0, :R, :] = dff
    cm_ref[0] = cm


def _stage1(sup, qry, nb, nqb, c):
    return pl.pallas_call(
        functools.partial(_stage1_body, nb),
        grid=(nb + 1,),
        in_specs=[
            pl.BlockSpec((1, c, N_WAY * MSP),
                         lambda t: (jnp.minimum(t, nb - 1) // nqb, 0, 0)),
            pl.BlockSpec((1, R, c), lambda t: (jnp.minimum(t, nb - 1), 0, 0)),
        ],
        out_specs=[
            pl.BlockSpec((1, RP, 1), lambda t: (jnp.maximum(t - 1, 0), 0, 0)),
            pl.BlockSpec((1, RP, 1), lambda t: (jnp.maximum(t - 1, 0), 0, 0)),
            pl.BlockSpec((1, R, N_WAY), lambda t: (jnp.maximum(t - 1, 0), 0, 0)),
        ],
        out_shape=[
            jax.ShapeDtypeStruct((nb, RP, 1), jnp.float32),
            jax.ShapeDtypeStruct((nb, RP, 1), jnp.float32),
            jax.ShapeDtypeStruct((nb, R, N_WAY), jnp.float32),
        ],
        scratch_shapes=[pltpu.VMEM((2, R, N_WAY * MSP), jnp.float32)],
        compiler_params=pltpu.CompilerParams(
            dimension_semantics=("arbitrary",)),
    )(sup, qry)


def _take16(vec, idx16):
    return lax.gather(
        vec, idx16[:, None],
        lax.GatherDimensionNumbers(offset_dims=(), collapsed_slice_dims=(0,),
                                   start_index_map=(0,)),
        (1,), mode=lax.GatherScatterMode.PROMISE_IN_BOUNDS)


def _sc_mask_body(nb, k_hbm, d_hbm, mask_hbm, k_v, d_v, m_v):
    wid = lax.axis_index("s") * 2 + lax.axis_index("c")
    nch = 7

    @pl.when(wid < nb)
    def _work():
        base = wid * RP
        pltpu.sync_copy(k_hbm.at[pl.ds(base, RP)], k_v)
        pltpu.sync_copy(d_hbm.at[pl.ds(base, RP)], d_v)
        lanef = lax.broadcasted_iota(jnp.int32, (16,), 0).astype(jnp.float32)

        def gloop(g, _):
            gb = g * M
            ki = [k_v[pl.ds(gb + c * 16, 16)] for c in range(nch)]
            di = [d_v[pl.ds(gb + c * 16, 16)] for c in range(nch)]
            idx = [lanef + float(c * 16) for c in range(nch)]
            beaten = [jnp.zeros((16,), jnp.float32) for _ in range(nch)]
            for jc in range(nch):
                for l in range(16 if jc < nch - 1 else M - (nch - 1) * 16):
                    kj = ki[jc][l]
                    dj = di[jc][l]
                    jf = float(jc * 16 + l)
                    for c in range(nch):
                        hit = (kj == ki[c]) & (
                            (dj > di[c]) | ((dj == di[c]) & (jf < idx[c])))
                        beaten[c] = jnp.where(hit, 1.0, beaten[c])
            for c in range(nch):
                keep = (beaten[c] < 0.5) & ((di[c] > 0.0) | (idx[c] == 0.0))
                m_v[pl.ds(gb + c * 16, 16)] = jnp.where(keep, 1.0, 0.0)
            return 0

        lax.fori_loop(0, QB, gloop, 0)
        pltpu.sync_copy(m_v, mask_hbm.at[pl.ds(base, RP)])


def _stage2(kflat, dflat, nb):
    mesh = plsc.VectorSubcoreMesh(core_axis_name="c", subcore_axis_name="s")
    fn = pl.kernel(
        functools.partial(_sc_mask_body, nb),
        out_type=jax.ShapeDtypeStruct((nb * RP,), jnp.float32),
        mesh=mesh,
        scratch_types=[
            pltpu.VMEM((RP,), jnp.float32),
            pltpu.VMEM((RP,), jnp.float32),
            pltpu.VMEM((RP,), jnp.float32),
        ],
    )
    return fn(kflat, dflat)


def _stage3_body(nb, cm_ref, mask_ref, lab_ref, out_ref):
    t = pl.program_id(0)
    cm = cm_ref[0]
    maskp = mask_ref[0, :R, :]
    z = cm * maskp
    gi = lax.broadcasted_iota(jnp.int32, (QB, R), 0)
    gj = lax.broadcasted_iota(jnp.int32, (QB, R), 1)
    sel = jnp.where(gi == gj // M, 1.0, 0.0)
    qvals = lax.dot_general(sel, z, (((1,), (0,)), ((), ())),
                            precision=lax.Precision.HIGHEST,
                            preferred_element_type=jnp.float32)
    lab = lab_ref[0]
    logits = qvals * (1.0 / TEMP)
    mx = jnp.max(logits, axis=1, keepdims=True)
    lse = jnp.log(jnp.sum(jnp.exp(logits - mx), axis=1, keepdims=True))
    i5b = lax.broadcasted_iota(jnp.int32, (QB, N_WAY), 1)
    picked = jnp.sum(jnp.where(i5b == lab, logits - mx, 0.0),
                     axis=1, keepdims=True)
    part = jnp.sum(lse - picked, keepdims=True)
    prev = jnp.where(t == 0, jnp.zeros((1, 1), jnp.float32), out_ref[...])
    out_ref[...] = prev + part


def _stage3(cm, maskb, labs, nb):
    return pl.pallas_call(
        functools.partial(_stage3_body, nb),
        grid=(nb,),
        in_specs=[
            pl.BlockSpec((1, R, N_WAY), lambda t: (t, 0, 0)),
            pl.BlockSpec((1, RP, 1), lambda t: (t, 0, 0)),
            pl.BlockSpec((1, QB, 1), lambda t: (t, 0, 0)),
        ],
        out_specs=pl.BlockSpec((1, 1), lambda t: (0, 0)),
        out_shape=jax.ShapeDtypeStruct((1, 1), jnp.float32),
        compiler_params=pltpu.CompilerParams(
            dimension_semantics=("arbitrary",)),
    )(cm, maskb, labs)


def kernel(support_xf, support_y, query_xf, query_y):
    b, s, c, h, w = support_xf.shape
    q = query_xf.shape[1]
    m = h * w
    nqb = q // QB
    nb = b * nqb
    nq = b * q

    sup = support_xf.reshape(b, N_WAY, K_SHOT, c, m)
    sup = jnp.transpose(sup, (0, 3, 1, 2, 4)).reshape(b, c, N_WAY, K_SHOT * m)
    sup = jnp.pad(sup, ((0, 0), (0, 0), (0, 0), (0, MSP - K_SHOT * m)))
    sup = sup.reshape(b, c, N_WAY * MSP)

    qry = jnp.transpose(query_xf.reshape(b, q, c, m), (0, 1, 3, 2))
    qry = qry.reshape(nb, R, c)

    labs = query_y.reshape(nb, QB, 1)

    kcol, dff, cm = _stage1(sup, qry, nb, nqb, c)

    maskf = _stage2(kcol.reshape(nb * RP), dff.reshape(nb * RP), nb)
    maskb = maskf.reshape(nb, RP, 1)

    total = _stage3(cm, maskb, labs, nb)
    return total[0, 0] / nq


# speedup vs baseline: 1.3087x; 1.3087x over previous
import functools

import jax
import jax.numpy as jnp
from jax import lax
from jax.experimental import pallas as pl
from jax.experimental.pallas import tpu as pltpu
from jax.experimental.pallas import tpu_sc as plsc

N_WAY = 5
K_SHOT = 5
TEMP = 2.0
MS = 500
MSP = 512
M = 100
QB = 5
R = QB * M
RP = 512
NW = 32


def _stage1_body(nb, sup_ref, qry_ref, kcol_ref, dff_ref, cm_ref, buf_ref):
    t = pl.program_id(0)
    wslot = t % 2
    rslot = (t + 1) % 2

    sup = sup_ref[0]
    snorm = jnp.sqrt(jnp.sum(sup * sup, axis=0, keepdims=True))
    sn = sup / (snorm + 1e-8)
    qv = qry_ref[0]
    qnorm = jnp.sqrt(jnp.sum(qv * qv, axis=1, keepdims=True))
    qn = qv / (qnorm + 1e-8)
    qh = qn.astype(jnp.bfloat16)
    ql = (qn - qh.astype(jnp.float32)).astype(jnp.bfloat16)
    sh = sn.astype(jnp.bfloat16)
    sl = (sn - sh.astype(jnp.float32)).astype(jnp.bfloat16)
    colpad = lax.broadcasted_iota(jnp.int32, (R, MSP), 1) >= MS
    cif = lax.broadcasted_iota(jnp.int32, (R, MSP), 1).astype(jnp.float32)

    dims = (((1,), (0,)), ((), ()))
    cm_cols = []
    am_cols = []
    for n in range(N_WAY):
        s_old = buf_ref[rslot, :, n * MSP:(n + 1) * MSP]
        rmax_n = jnp.max(s_old, axis=1, keepdims=True)
        am_n = jnp.min(jnp.where(s_old == rmax_n, cif, float(MSP)),
                       axis=1, keepdims=True)
        cm_cols.append(rmax_n)
        am_cols.append(am_n)
        csl = slice(n * MSP, (n + 1) * MSP)
        s_new = (lax.dot_general(qh, sh[:, csl], dims,
                                 preferred_element_type=jnp.float32)
                 + (lax.dot_general(qh, sl[:, csl], dims,
                                    preferred_element_type=jnp.float32)
                    + lax.dot_general(ql, sh[:, csl], dims,
                                      preferred_element_type=jnp.float32)))
        buf_ref[wslot, :, n * MSP:(n + 1) * MSP] = \
            jnp.where(colpad, -2.0, s_new)

    cm = jnp.concatenate(cm_cols, axis=1)
    t1 = jnp.max(cm, axis=1, keepdims=True)
    i5 = lax.broadcasted_iota(jnp.int32, (R, N_WAY), 1).astype(jnp.float32)
    first = jnp.min(jnp.where(cm == t1, i5, float(N_WAY)),
                    axis=1, keepdims=True)
    t2 = jnp.max(jnp.where(i5 == first, -3.0, cm), axis=1, keepdims=True)
    dff = t1 - t2

    kcol = jnp.full((R, 1), jnp.float32(N_WAY * MSP))
    for n in range(N_WAY):
        cand = jnp.where(cm[:, n:n + 1] == t1, am_cols[n] + float(n * MSP),
                         jnp.float32(N_WAY * MSP))
        kcol = jnp.minimum(kcol, cand)

    kcol_ref[0, :R, :] = kcol
    dff_ref[0, :R, :] = dff
    cm_ref[0] = cm


def _stage1(sup, qry, nb, nqb, c):
    return pl.pallas_call(
        functools.partial(_stage1_body, nb),
        grid=(nb + 1,),
        in_specs=[
            pl.BlockSpec((1, c, N_WAY * MSP),
                         lambda t: (jnp.minimum(t, nb - 1) // nqb, 0, 0)),
            pl.BlockSpec((1, R, c), lambda t: (jnp.minimum(t, nb - 1), 0, 0)),
        ],
        out_specs=[
            pl.BlockSpec((1, RP, 1), lambda t: (jnp.maximum(t - 1, 0), 0, 0)),
            pl.BlockSpec((1, RP, 1), lambda t: (jnp.maximum(t - 1, 0), 0, 0)),
            pl.BlockSpec((1, R, N_WAY), lambda t: (jnp.maximum(t - 1, 0), 0, 0)),
        ],
        out_shape=[
            jax.ShapeDtypeStruct((nb, RP, 1), jnp.float32),
            jax.ShapeDtypeStruct((nb, RP, 1), jnp.float32),
            jax.ShapeDtypeStruct((nb, R, N_WAY), jnp.float32),
        ],
        scratch_shapes=[pltpu.VMEM((2, R, N_WAY * MSP), jnp.float32)],
        compiler_params=pltpu.CompilerParams(
            dimension_semantics=("arbitrary",)),
    )(sup, qry)


def _take16(vec, idx16):
    return lax.gather(
        vec, idx16[:, None],
        lax.GatherDimensionNumbers(offset_dims=(), collapsed_slice_dims=(0,),
                                   start_index_map=(0,)),
        (1,), mode=lax.GatherScatterMode.PROMISE_IN_BOUNDS)


def _sc_mask_body(nb, k_hbm, d_hbm, mask_hbm, k_v, d_v, m_v):
    wid = lax.axis_index("s") * 2 + lax.axis_index("c")
    nch = 7

    @pl.when(wid < nb)
    def _work():
        base = wid * RP
        pltpu.sync_copy(k_hbm.at[pl.ds(base, RP)], k_v)
        pltpu.sync_copy(d_hbm.at[pl.ds(base, RP)], d_v)
        lanef = lax.broadcasted_iota(jnp.int32, (16,), 0).astype(jnp.float32)

        def gloop(g, _):
            gb = g * M
            ki = [k_v[pl.ds(gb + c * 16, 16)] for c in range(nch)]
            di = [d_v[pl.ds(gb + c * 16, 16)] for c in range(nch)]
            idx = [lanef + float(c * 16) for c in range(nch)]
            beaten = [jnp.zeros((16,), jnp.float32) for _ in range(nch)]
            for jc in range(nch):
                for l in range(16 if jc < nch - 1 else M - (nch - 1) * 16):
                    kj = ki[jc][l]
                    dj = di[jc][l]
                    jf = float(jc * 16 + l)
                    for c in range(nch):
                        hit = (kj == ki[c]) & (
                            (dj > di[c]) | ((dj == di[c]) & (jf < idx[c])))
                        beaten[c] = jnp.where(hit, 1.0, beaten[c])
            for c in range(nch):
                keep = (beaten[c] < 0.5) & ((di[c] > 0.0) | (idx[c] == 0.0))
                m_v[pl.ds(gb + c * 16, 16)] = jnp.where(keep, 1.0, 0.0)
            return 0

        lax.fori_loop(0, QB, gloop, 0)
        pltpu.sync_copy(m_v, mask_hbm.at[pl.ds(base, RP)])


def _stage2(kflat, dflat, nb):
    mesh = plsc.VectorSubcoreMesh(core_axis_name="c", subcore_axis_name="s")
    fn = pl.kernel(
        functools.partial(_sc_mask_body, nb),
        out_type=jax.ShapeDtypeStruct((nb * RP,), jnp.float32),
        mesh=mesh,
        scratch_types=[
            pltpu.VMEM((RP,), jnp.float32),
            pltpu.VMEM((RP,), jnp.float32),
            pltpu.VMEM((RP,), jnp.float32),
        ],
    )
    return fn(kflat, dflat)


def _stage3_body(nb, cm_ref, mask_ref, lab_ref, out_ref):
    t = pl.program_id(0)
    cm = cm_ref[0]
    maskp = mask_ref[0, :R, :]
    z = cm * maskp
    gi = lax.broadcasted_iota(jnp.int32, (QB, R), 0)
    gj = lax.broadcasted_iota(jnp.int32, (QB, R), 1)
    sel = jnp.where(gi == gj // M, 1.0, 0.0)
    qvals = lax.dot_general(sel, z, (((1,), (0,)), ((), ())),
                            precision=lax.Precision.HIGHEST,
                            preferred_element_type=jnp.float32)
    lab = lab_ref[0]
    logits = qvals * (1.0 / TEMP)
    mx = jnp.max(logits, axis=1, keepdims=True)
    lse = jnp.log(jnp.sum(jnp.exp(logits - mx), axis=1, keepdims=True))
    i5b = lax.broadcasted_iota(jnp.int32, (QB, N_WAY), 1)
    picked = jnp.sum(jnp.where(i5b == lab, logits - mx, 0.0),
                     axis=1, keepdims=True)
    part = jnp.sum(lse - picked, keepdims=True)
    prev = jnp.where(t == 0, jnp.zeros((1, 1), jnp.float32), out_ref[...])
    out_ref[...] = prev + part


def _stage3(cm, maskb, labs, nb):
    return pl.pallas_call(
        functools.partial(_stage3_body, nb),
        grid=(nb,),
        in_specs=[
            pl.BlockSpec((1, R, N_WAY), lambda t: (t, 0, 0)),
            pl.BlockSpec((1, RP, 1), lambda t: (t, 0, 0)),
            pl.BlockSpec((1, QB, 1), lambda t: (t, 0, 0)),
        ],
        out_specs=pl.BlockSpec((1, 1), lambda t: (0, 0)),
        out_shape=jax.ShapeDtypeStruct((1, 1), jnp.float32),
        compiler_params=pltpu.CompilerParams(
            dimension_semantics=("arbitrary",)),
    )(cm, maskb, labs)


def kernel(support_xf, support_y, query_xf, query_y):
    b, s, c, h, w = support_xf.shape
    q = query_xf.shape[1]
    m = h * w
    nqb = q // QB
    nb = b * nqb
    nq = b * q

    sup = support_xf.reshape(b, N_WAY, K_SHOT, c, m)
    sup = jnp.transpose(sup, (0, 3, 1, 2, 4)).reshape(b, c, N_WAY, K_SHOT * m)
    sup = jnp.pad(sup, ((0, 0), (0, 0), (0, 0), (0, MSP - K_SHOT * m)))
    sup = sup.reshape(b, c, N_WAY * MSP)

    qry = jnp.transpose(query_xf.reshape(b, q, c, m), (0, 1, 3, 2))
    qry = qry.reshape(nb, R, c)

    labs = query_y.reshape(nb, QB, 1)

    kcol, dff, cm = _stage1(sup, qry, nb, nqb, c)

    maskf = _stage2(kcol.reshape(nb * RP), dff.reshape(nb * RP), nb)
    maskb = maskf.reshape(nb, RP, 1)

    total = _stage3(cm, maskb, labs, nb)
    return total[0, 0] / nq


# speedup vs baseline: 1.3176x; 1.0068x over previous
import functools

import jax
import jax.numpy as jnp
from jax import lax
from jax.experimental import pallas as pl
from jax.experimental.pallas import tpu as pltpu
from jax.experimental.pallas import tpu_sc as plsc

N_WAY = 5
K_SHOT = 5
TEMP = 2.0
MS = 500
MSP = 512
M = 100
QB = 5
R = QB * M
RP = 512


def _stage1_body(nb, sup_ref, qry_ref, kcol_ref, dff_ref, cm_ref, buf_ref):
    t = pl.program_id(0)
    wslot = t % 2
    rslot = (t + 1) % 2

    sup = sup_ref[0]
    snorm = jnp.sqrt(jnp.sum(sup * sup, axis=0, keepdims=True))
    sn = sup / (snorm + 1e-8)
    qv = qry_ref[0]
    qnorm = jnp.sqrt(jnp.sum(qv * qv, axis=1, keepdims=True))
    qn = qv / (qnorm + 1e-8)
    qh = qn.astype(jnp.bfloat16)
    ql = (qn - qh.astype(jnp.float32)).astype(jnp.bfloat16)
    sh = sn.astype(jnp.bfloat16)
    sl = (sn - sh.astype(jnp.float32)).astype(jnp.bfloat16)
    colpad = lax.broadcasted_iota(jnp.int32, (R, MSP), 1) >= MS
    cif = lax.broadcasted_iota(jnp.int32, (R, MSP), 1).astype(jnp.float32)

    dims = (((1,), (0,)), ((), ()))
    cm_cols = []
    am_cols = []
    for n in range(N_WAY):
        s_old = buf_ref[rslot, :, n * MSP:(n + 1) * MSP]
        rmax_n = jnp.max(s_old, axis=1, keepdims=True)
        am_n = jnp.min(jnp.where(s_old == rmax_n, cif, float(MSP)),
                       axis=1, keepdims=True)
        cm_cols.append(rmax_n)
        am_cols.append(am_n)
        csl = slice(n * MSP, (n + 1) * MSP)
        s_new = (lax.dot_general(qh, sh[:, csl], dims,
                                 preferred_element_type=jnp.float32)
                 + (lax.dot_general(qh, sl[:, csl], dims,
                                    preferred_element_type=jnp.float32)
                    + lax.dot_general(ql, sh[:, csl], dims,
                                      preferred_element_type=jnp.float32)))
        buf_ref[wslot, :, n * MSP:(n + 1) * MSP] = \
            jnp.where(colpad, -2.0, s_new)

    cm = jnp.concatenate(cm_cols, axis=1)
    t1 = jnp.max(cm, axis=1, keepdims=True)
    i5 = lax.broadcasted_iota(jnp.int32, (R, N_WAY), 1).astype(jnp.float32)
    first = jnp.min(jnp.where(cm == t1, i5, float(N_WAY)),
                    axis=1, keepdims=True)
    t2 = jnp.max(jnp.where(i5 == first, -3.0, cm), axis=1, keepdims=True)
    dff = t1 - t2

    kcol = jnp.full((R, 1), jnp.float32(N_WAY * MSP))
    for n in range(N_WAY):
        cand = jnp.where(cm[:, n:n + 1] == t1, am_cols[n] + float(n * MSP),
                         jnp.float32(N_WAY * MSP))
        kcol = jnp.minimum(kcol, cand)

    kcol_ref[0, :R, :] = kcol
    dff_ref[0, :R, :] = dff
    cm_ref[0] = cm


def _stage1(sup, qry, nb, nqb, c):
    return pl.pallas_call(
        functools.partial(_stage1_body, nb),
        grid=(nb + 1,),
        in_specs=[
            pl.BlockSpec((1, c, N_WAY * MSP),
                         lambda t: (jnp.minimum(t, nb - 1) // nqb, 0, 0)),
            pl.BlockSpec((1, R, c), lambda t: (jnp.minimum(t, nb - 1), 0, 0)),
        ],
        out_specs=[
            pl.BlockSpec((1, RP, 1), lambda t: (jnp.maximum(t - 1, 0), 0, 0)),
            pl.BlockSpec((1, RP, 1), lambda t: (jnp.maximum(t - 1, 0), 0, 0)),
            pl.BlockSpec((1, R, N_WAY), lambda t: (jnp.maximum(t - 1, 0), 0, 0)),
        ],
        out_shape=[
            jax.ShapeDtypeStruct((nb, RP, 1), jnp.float32),
            jax.ShapeDtypeStruct((nb, RP, 1), jnp.float32),
            jax.ShapeDtypeStruct((nb, R, N_WAY), jnp.float32),
        ],
        scratch_shapes=[pltpu.VMEM((2, R, N_WAY * MSP), jnp.float32)],
        compiler_params=pltpu.CompilerParams(
            dimension_semantics=("arbitrary",)),
    )(sup, qry)


def _sc_mask_body(nb, k_hbm, d_hbm, mask_hbm, k_v, d_v, m_v):
    wid = lax.axis_index("s") * 2 + lax.axis_index("c")
    nch = 7

    @pl.when(wid < nb)
    def _work():
        base = wid * RP
        pltpu.sync_copy(k_hbm.at[pl.ds(base, RP)], k_v)
        pltpu.sync_copy(d_hbm.at[pl.ds(base, RP)], d_v)
        lanef = lax.broadcasted_iota(jnp.int32, (16,), 0).astype(jnp.float32)

        def gloop(g, _):
            gb = g * M
            ki = [k_v[pl.ds(gb + c * 16, 16)] for c in range(nch)]
            di = [d_v[pl.ds(gb + c * 16, 16)] for c in range(nch)]
            idx = [lanef + float(c * 16) for c in range(nch)]
            beaten = [jnp.zeros((16,), jnp.float32) for _ in range(nch)]
            for jc in range(nch):
                for l in range(16 if jc < nch - 1 else M - (nch - 1) * 16):
                    kj = ki[jc][l]
                    dj = di[jc][l]
                    jf = float(jc * 16 + l)
                    for c in range(nch):
                        hit = (kj == ki[c]) & (
                            (dj > di[c]) | ((dj == di[c]) & (jf < idx[c])))
                        beaten[c] = jnp.where(hit, 1.0, beaten[c])
            for c in range(nch):
                keep = (beaten[c] < 0.5) & ((di[c] > 0.0) | (idx[c] == 0.0))
                m_v[pl.ds(gb + c * 16, 16)] = jnp.where(keep, 1.0, 0.0)
            return 0

        lax.fori_loop(0, QB, gloop, 0)
        pltpu.sync_copy(m_v, mask_hbm.at[pl.ds(base, RP)])


def _stage2(kflat, dflat, nb):
    mesh = plsc.VectorSubcoreMesh(core_axis_name="c", subcore_axis_name="s")
    fn = pl.kernel(
        functools.partial(_sc_mask_body, nb),
        out_type=jax.ShapeDtypeStruct((nb * RP,), jnp.float32),
        mesh=mesh,
        scratch_types=[
            pltpu.VMEM((RP,), jnp.float32),
            pltpu.VMEM((RP,), jnp.float32),
            pltpu.VMEM((RP,), jnp.float32),
        ],
    )
    return fn(kflat, dflat)


def _stage3_body(nb, cm_ref, mask_ref, lab_ref, out_ref):
    t = pl.program_id(0)
    cm = cm_ref[0]
    maskp = mask_ref[0, :R, :]
    z = cm * maskp
    gi = lax.broadcasted_iota(jnp.int32, (QB, R), 0)
    gj = lax.broadcasted_iota(jnp.int32, (QB, R), 1)
    sel = jnp.where(gi == gj // M, 1.0, 0.0)
    qvals = lax.dot_general(sel, z, (((1,), (0,)), ((), ())),
                            precision=lax.Precision.HIGHEST,
                            preferred_element_type=jnp.float32)
    lab = lab_ref[0]
    logits = qvals * (1.0 / TEMP)
    mx = jnp.max(logits, axis=1, keepdims=True)
    lse = jnp.log(jnp.sum(jnp.exp(logits - mx), axis=1, keepdims=True))
    i5b = lax.broadcasted_iota(jnp.int32, (QB, N_WAY), 1)
    picked = jnp.sum(jnp.where(i5b == lab, logits - mx, 0.0),
                     axis=1, keepdims=True)
    part = jnp.sum(lse - picked, keepdims=True)
    prev = jnp.where(t == 0, jnp.zeros((1, 1), jnp.float32), out_ref[...])
    out_ref[...] = prev + part


def _stage3(cm, maskb, labs, nb):
    return pl.pallas_call(
        functools.partial(_stage3_body, nb),
        grid=(nb,),
        in_specs=[
            pl.BlockSpec((1, R, N_WAY), lambda t: (t, 0, 0)),
            pl.BlockSpec((1, RP, 1), lambda t: (t, 0, 0)),
            pl.BlockSpec((1, QB, 1), lambda t: (t, 0, 0)),
        ],
        out_specs=pl.BlockSpec((1, 1), lambda t: (0, 0)),
        out_shape=jax.ShapeDtypeStruct((1, 1), jnp.float32),
        compiler_params=pltpu.CompilerParams(
            dimension_semantics=("arbitrary",)),
    )(cm, maskb, labs)


def kernel(support_xf, support_y, query_xf, query_y):
    b, s, c, h, w = support_xf.shape
    q = query_xf.shape[1]
    m = h * w
    nqb = q // QB
    nb = b * nqb
    nq = b * q

    sup = support_xf.reshape(b, N_WAY, K_SHOT, c, m)
    sup = jnp.transpose(sup, (0, 3, 1, 2, 4)).reshape(b, c, N_WAY, K_SHOT * m)
    sup = jnp.pad(sup, ((0, 0), (0, 0), (0, 0), (0, MSP - K_SHOT * m)))
    sup = sup.reshape(b, c, N_WAY * MSP)

    qry = jnp.transpose(query_xf.reshape(b, q, c, m), (0, 1, 3, 2))
    qry = qry.reshape(nb, R, c)

    labs = query_y.reshape(nb, QB, 1)

    kcol, dff, cm = _stage1(sup, qry, nb, nqb, c)

    maskf = _stage2(kcol.reshape(nb * RP), dff.reshape(nb * RP), nb)
    maskb = maskf.reshape(nb, RP, 1)

    total = _stage3(cm, maskb, labs, nb)
    return total[0, 0] / nq
